# baseline (speedup 1.0000x reference)
import functools

import jax
import jax.numpy as jnp
from jax import lax
from jax.experimental import pallas as pl
from jax.experimental.pallas import tpu as pltpu
from jax.experimental.pallas import tpu_sc as plsc

N = 1600000
SEG = 50000
NC, NS, L = 2, 16, 16
NW = NC * NS
CHUNK = N // NW
BLK = 10000
NBLK = CHUNK // BLK
GRP = BLK // L
SLICE = 3136
PAD_SEG = NS * SLICE
HALF = N // 2
FB = 2048
FGRP = FB // L
W = 8192
SHSEG = PAD_SEG + W + 8

_mesh = plsc.VectorSubcoreMesh(core_axis_name="c", subcore_axis_name="s")
_params = pltpu.CompilerParams(needs_layout_passes=False)


@functools.partial(
    pl.kernel,
    out_type=jax.ShapeDtypeStruct((N,), jnp.float32),
    mesh=_mesh,
    compiler_params=_params,
    scratch_types=[
        pltpu.VMEM((BLK,), jnp.float32),
        pltpu.VMEM((BLK,), jnp.float32),
        pltpu.VMEM((BLK,), jnp.int32),
        pltpu.VMEM((BLK,), jnp.float32),
        pltpu.VMEM((BLK,), jnp.float32),
        pltpu.VMEM((BLK,), jnp.int32),
        pltpu.VMEM((BLK,), jnp.int32),
        pltpu.VMEM((BLK,), jnp.float32),
        pltpu.VMEM((W,), jnp.float32),
        pltpu.VMEM((W,), jnp.float32),
        pltpu.VMEM((W,), jnp.int32),
        pltpu.VMEM((W + 8,), jnp.float32),
        pltpu.VMEM((16,), jnp.int32),
        pltpu.VMEM((16,), jnp.float32),
        pltpu.VMEM((16,), jnp.float32),
        pltpu.VMEM_SHARED((SHSEG,), jnp.float32),
        pltpu.VMEM_SHARED((SHSEG,), jnp.float32),
        pltpu.SemaphoreType.DMA,
        pltpu.SemaphoreType.DMA,
        pltpu.SemaphoreType.DMA,
        pltpu.SemaphoreType.DMA,
    ],
)
def _fused(e_hbm, h_hbm, fc_hbm, sid_hbm, out_hbm, ev0, hv0, sidv0,
           ev1, hv1, sidv1, fcv, pmbuf, accAw, accBw, idxb, pmw,
           fixidx, fixA, fixB, accA, accB, sin0, sin1, ssc0, ssc1):
    c = lax.axis_index("c")
    s = lax.axis_index("s")
    wid = c * NS + s
    base = wid * CHUNK
    iota = lax.iota(jnp.int32, 16)
    bufs = [(ev0, hv0, sidv0, sin0, ssc0),
            (ev1, hv1, sidv1, sin1, ssc1)]

    pltpu.sync_copy(sid_hbm.at[pl.ds(base, 8)], fixidx.at[pl.ds(0, 8)])
    pltpu.sync_copy(sid_hbm.at[pl.ds(base + CHUNK - 8, 8)],
                    fixidx.at[pl.ds(8, 8)])
    bv0 = fixidx[pl.ds(0, 16)]
    first = bv0[0]
    last = bv0[15]
    local = (last - first) < W
    first_v = jnp.full((16,), first, jnp.int32)

    @plsc.parallel_loop(0, W // 16, unroll=4)
    def _zw(j):
        d = pl.ds(j * 16, 16)
        z = jnp.zeros((16,), jnp.float32)
        accAw[d] = z
        accBw[d] = z

    @plsc.parallel_loop(0, SLICE // 16, unroll=4)
    def _zfill(j):
        ev0[pl.ds(j * 16, 16)] = jnp.zeros((16,), jnp.float32)
    pltpu.sync_copy(ev0.at[pl.ds(0, SLICE)], accA.at[pl.ds(s * SLICE, SLICE)])
    pltpu.sync_copy(ev0.at[pl.ds(0, SLICE)], accB.at[pl.ds(s * SLICE, SLICE)])
    plsc.subcore_barrier()

    def _start_in(blk):
        ev, hv, sidv, sin, _ = bufs[blk % 2]
        st = base + blk * BLK
        return [pltpu.async_copy(e_hbm.at[pl.ds(st, BLK)], ev, sin),
                pltpu.async_copy(h_hbm.at[pl.ds(st, BLK)], hv, sin),
                pltpu.async_copy(sid_hbm.at[pl.ds(st, BLK)], sidv, sin)]

    def _start_fc(blk):
        st = base + blk * BLK
        return pltpu.async_copy(fc_hbm.at[pl.ds(st, BLK)], fcv, sin0)

    in_cps = {0: _start_in(0)}
    fc_cp = _start_fc(0)
    for blk in range(NBLK):
        ev, hv, sidv, sin, ssc = bufs[blk % 2]
        for cp in in_cps.pop(blk):
            cp.wait()
        fc_cp.wait()
        if blk + 1 < NBLK:
            in_cps[blk + 1] = _start_in(blk + 1)

        @pl.when(local)
        def _local_acc():
            def _grp(j, _):
                d = pl.ds(j * 16, 16)
                inv = 1.0 / hv[d]
                val = inv * ev[d] + fcv[d].astype(jnp.float32)
                widx = sidv[d] - first_v
                plsc.addupdate_scatter(accAw, [widx], val)
                plsc.addupdate_scatter(accBw, [widx], inv)
                return 0
            lax.fori_loop(0, GRP, _grp, 0)

        @pl.when(jnp.logical_not(local))
        def _spill_acc():
            @plsc.parallel_loop(0, GRP, unroll=5)
            def _grp(j):
                d = pl.ds(j * 16, 16)
                inv = 1.0 / hv[d]
                ev[d] = inv * ev[d] + fcv[d].astype(jnp.float32)
                hv[d] = inv
            pltpu.sync_copy(ev, accA.at[sidv], add=True)
            pltpu.sync_copy(hv, accB.at[sidv], add=True)

        if blk + 1 < NBLK:
            fc_cp = _start_fc(blk + 1)

    @pl.when(local)
    def _merge():
        @plsc.parallel_loop(0, W // 16, unroll=4)
        def _ib(j):
            idxb[pl.ds(j * 16, 16)] = first_v + j * 16 + iota
        pltpu.sync_copy(accAw, accA.at[idxb], add=True)
        pltpu.sync_copy(accBw, accB.at[idxb], add=True)

    plsc.subcore_barrier()

    @pl.when(s == 0)
    def _fixup():
        pltpu.sync_copy(sid_hbm.at[pl.ds(HALF - 8, 16)], fixidx)
        bv = fixidx[pl.ds(0, 16)]
        sid_l = bv[7]
        sid_r = bv[8]

        @pl.when(sid_l == sid_r)
        def _straddle():
            sv = jnp.full((16,), sid_l, jnp.int32)
            fwd = c == 0

            def _cond(carry):
                t, go, _, _ = carry
                return go & (t < HALF // FB)

            def _body(carry):
                t, go, vA, vB = carry
                off = jnp.where(fwd, HALF + t * FB, HALF - (t + 1) * FB)
                pltpu.sync_copy(e_hbm.at[pl.ds(off, FB)], ev0.at[pl.ds(0, FB)])
                pltpu.sync_copy(h_hbm.at[pl.ds(off, FB)], hv0.at[pl.ds(0, FB)])
                pltpu.sync_copy(fc_hbm.at[pl.ds(off, FB)], fcv.at[pl.ds(0, FB)])
                pltpu.sync_copy(sid_hbm.at[pl.ds(off, FB)], sidv0.at[pl.ds(0, FB)])

                def _fgrp(j, fcarry):
                    fvA, fvB, nmatch = fcarry
                    d = pl.ds(j * 16, 16)
                    m = sidv0[d] == sv
                    inv = 1.0 / hv0[d]
                    val = inv * ev0[d] + fcv[d].astype(jnp.float32)
                    zf = jnp.zeros((16,), jnp.float32)
                    fvA = fvA + jnp.where(m, val, zf)
                    fvB = fvB + jnp.where(m, inv, zf)
                    nmatch = nmatch + jnp.sum(m.astype(jnp.int32))
                    return fvA, fvB, nmatch

                vA, vB, nmatch = lax.fori_loop(
                    0, FGRP, _fgrp, (vA, vB, jnp.int32(0)))
                return t + 1, go & (nmatch == FB), vA, vB

            zf16 = jnp.zeros((16,), jnp.float32)
            _, _, vA, vB = lax.while_loop(
                _cond, _body, (jnp.int32(0), jnp.bool_(True), zf16, zf16))

            lane = lax.iota(jnp.int32, 16)
            firstl = (lane == 0).astype(jnp.float32)
            fixidx[:] = sv
            fixA[:] = jnp.sum(vA) * firstl
            fixB[:] = jnp.sum(vB) * firstl
            pltpu.sync_copy(fixA, accA.at[fixidx], add=True)
            pltpu.sync_copy(fixB, accB.at[fixidx], add=True)

    plsc.subcore_barrier()

    sl = pl.ds(s * SLICE, SLICE)
    pltpu.sync_copy(accA.at[sl], ev0.at[pl.ds(0, SLICE)])
    pltpu.sync_copy(accB.at[sl], hv0.at[pl.ds(0, SLICE)])

    @plsc.parallel_loop(0, SLICE // 16, unroll=4)
    def _pm(j):
        d = pl.ds(j * 16, 16)
        ev1[d] = ev0[d] / hv0[d]
    pltpu.sync_copy(ev1.at[pl.ds(0, SLICE)], accA.at[sl])
    plsc.subcore_barrier()

    wb = (first // 8) * 8
    wb_v = jnp.full((16,), wb, jnp.int32)

    @pl.when(local)
    def _pull_pm():
        pltpu.sync_copy(accA.at[pl.ds(wb, W + 8)], pmw)

    in_cps = {0: _start_in(0)}
    st_cps = {}
    for blk in range(NBLK):
        ev, hv, sidv, sin, ssc = bufs[blk % 2]
        for cp in in_cps.pop(blk):
            cp.wait()
        if blk + 1 < NBLK:
            if blk - 1 >= 0:
                for cp in st_cps.pop(blk - 1):
                    cp.wait()
            in_cps[blk + 1] = _start_in(blk + 1)

        @pl.when(local)
        def _local_out():
            @plsc.parallel_loop(0, GRP, unroll=5)
            def _out(j):
                d = pl.ds(j * 16, 16)
                pmg = plsc.load_gather(pmw, [sidv[d] - wb_v])
                inv = 1.0 / hv[d]
                ev[d] = inv * (pmg - ev[d])

        @pl.when(jnp.logical_not(local))
        def _spill_out():
            pltpu.sync_copy(accA.at[sidv], pmbuf)

            @plsc.parallel_loop(0, GRP, unroll=5)
            def _out(j):
                d = pl.ds(j * 16, 16)
                inv = 1.0 / hv[d]
                ev[d] = inv * (pmbuf[d] - ev[d])

        st = base + blk * BLK
        st_cps[blk] = [pltpu.async_copy(ev, out_hbm.at[pl.ds(st, BLK)], ssc)]
    for blk in sorted(st_cps):
        for cp in st_cps.pop(blk):
            cp.wait()


@jax.jit
def kernel(x, formal_charge, segment_ids):
    sid = segment_ids.astype(jnp.int32)
    fc = formal_charge.astype(jnp.int32)
    e = x[:, 0]
    h = x[:, 1]
    charges = _fused(e, h, fc, sid)
    return charges.reshape(-1, 1)


# speedup vs baseline: 1.1143x; 1.1143x over previous
import functools

import jax
import jax.numpy as jnp
from jax import lax
from jax.experimental import pallas as pl
from jax.experimental.pallas import tpu as pltpu
from jax.experimental.pallas import tpu_sc as plsc

N = 1600000
SEG = 50000
NC, NS, L = 2, 16, 16
NW = NC * NS
CHUNK = N // NW
BLK = 10000
NBLK = CHUNK // BLK
GRP = BLK // L
SLICE = 3136
PAD_SEG = NS * SLICE
HALF = N // 2
FB = 2048
FGRP = FB // L
W = 8192
SHSEG = PAD_SEG + W + 8

_mesh = plsc.VectorSubcoreMesh(core_axis_name="c", subcore_axis_name="s")
_params = pltpu.CompilerParams(needs_layout_passes=False)


@functools.partial(
    pl.kernel,
    out_type=jax.ShapeDtypeStruct((N,), jnp.float32),
    mesh=_mesh,
    compiler_params=_params,
    scratch_types=[
        pltpu.VMEM((BLK,), jnp.float32),
        pltpu.VMEM((BLK,), jnp.float32),
        pltpu.VMEM((BLK,), jnp.int32),
        pltpu.VMEM((BLK,), jnp.float32),
        pltpu.VMEM((BLK,), jnp.float32),
        pltpu.VMEM((BLK,), jnp.int32),
        pltpu.VMEM((BLK,), jnp.int32),
        pltpu.VMEM((BLK,), jnp.float32),
        pltpu.VMEM((W,), jnp.float32),
        pltpu.VMEM((W,), jnp.float32),
        pltpu.VMEM((W,), jnp.int32),
        pltpu.VMEM((W + 8,), jnp.float32),
        pltpu.VMEM((16,), jnp.int32),
        pltpu.VMEM((16,), jnp.float32),
        pltpu.VMEM((16,), jnp.float32),
        pltpu.VMEM_SHARED((SHSEG,), jnp.float32),
        pltpu.VMEM_SHARED((SHSEG,), jnp.float32),
        pltpu.SemaphoreType.DMA,
        pltpu.SemaphoreType.DMA,
        pltpu.SemaphoreType.DMA,
        pltpu.SemaphoreType.DMA,
    ],
)
def _fused(e_hbm, h_hbm, fc_hbm, sid_hbm, out_hbm, ev0, hv0, sidv0,
           ev1, hv1, sidv1, fcv, pmbuf, accAw, accBw, idxb, pmw,
           fixidx, fixA, fixB, accA, accB, sin0, sin1, ssc0, ssc1):
    c = lax.axis_index("c")
    s = lax.axis_index("s")
    wid = c * NS + s
    base = wid * CHUNK
    iota = lax.iota(jnp.int32, 16)
    bufs = [(ev0, hv0, sidv0, sin0, ssc0),
            (ev1, hv1, sidv1, sin1, ssc1)]

    pltpu.sync_copy(sid_hbm.at[pl.ds(base, 8)], fixidx.at[pl.ds(0, 8)])
    pltpu.sync_copy(sid_hbm.at[pl.ds(base + CHUNK - 8, 8)],
                    fixidx.at[pl.ds(8, 8)])
    bv0 = fixidx[pl.ds(0, 16)]
    first = bv0[0]
    last = bv0[15]
    local = (last - first) < W
    first_v = jnp.full((16,), first, jnp.int32)

    @plsc.parallel_loop(0, W // 16, unroll=4)
    def _zw(j):
        d = pl.ds(j * 16, 16)
        z = jnp.zeros((16,), jnp.float32)
        accAw[d] = z
        accBw[d] = z

    @plsc.parallel_loop(0, SLICE // 16, unroll=4)
    def _zfill(j):
        ev0[pl.ds(j * 16, 16)] = jnp.zeros((16,), jnp.float32)
    pltpu.sync_copy(ev0.at[pl.ds(0, SLICE)], accA.at[pl.ds(s * SLICE, SLICE)])
    pltpu.sync_copy(ev0.at[pl.ds(0, SLICE)], accB.at[pl.ds(s * SLICE, SLICE)])
    plsc.subcore_barrier()

    def _start_in(blk):
        ev, hv, sidv, sin, _ = bufs[blk % 2]
        st = base + blk * BLK
        return [pltpu.async_copy(e_hbm.at[pl.ds(st, BLK)], ev, sin),
                pltpu.async_copy(h_hbm.at[pl.ds(st, BLK)], hv, sin),
                pltpu.async_copy(sid_hbm.at[pl.ds(st, BLK)], sidv, sin)]

    def _start_fc(blk):
        st = base + blk * BLK
        return pltpu.async_copy(fc_hbm.at[pl.ds(st, BLK)], fcv, sin0)

    in_cps = {0: _start_in(0)}
    fc_cp = _start_fc(0)
    for blk in range(NBLK):
        ev, hv, sidv, sin, ssc = bufs[blk % 2]
        for cp in in_cps.pop(blk):
            cp.wait()
        fc_cp.wait()
        if blk + 1 < NBLK:
            in_cps[blk + 1] = _start_in(blk + 1)

        @pl.when(local)
        def _local_acc():
            @plsc.parallel_loop(0, GRP, unroll=5)
            def _grp(j):
                d = pl.ds(j * 16, 16)
                inv = 1.0 / hv[d]
                val = inv * ev[d] + fcv[d].astype(jnp.float32)
                widx = sidv[d] - first_v
                plsc.addupdate_scatter(accAw, [widx], val)
                plsc.addupdate_scatter(accBw, [widx], inv)

        @pl.when(jnp.logical_not(local))
        def _spill_acc():
            @plsc.parallel_loop(0, GRP, unroll=5)
            def _grp(j):
                d = pl.ds(j * 16, 16)
                inv = 1.0 / hv[d]
                ev[d] = inv * ev[d] + fcv[d].astype(jnp.float32)
                hv[d] = inv
            pltpu.sync_copy(ev, accA.at[sidv], add=True)
            pltpu.sync_copy(hv, accB.at[sidv], add=True)

        if blk + 1 < NBLK:
            fc_cp = _start_fc(blk + 1)

    @pl.when(local)
    def _merge():
        @plsc.parallel_loop(0, W // 16, unroll=4)
        def _ib(j):
            idxb[pl.ds(j * 16, 16)] = first_v + j * 16 + iota
        pltpu.sync_copy(accAw, accA.at[idxb], add=True)
        pltpu.sync_copy(accBw, accB.at[idxb], add=True)

    plsc.subcore_barrier()

    @pl.when(s == 0)
    def _fixup():
        pltpu.sync_copy(sid_hbm.at[pl.ds(HALF - 8, 16)], fixidx)
        bv = fixidx[pl.ds(0, 16)]
        sid_l = bv[7]
        sid_r = bv[8]

        @pl.when(sid_l == sid_r)
        def _straddle():
            sv = jnp.full((16,), sid_l, jnp.int32)
            fwd = c == 0

            def _cond(carry):
                t, go, _, _ = carry
                return go & (t < HALF // FB)

            def _body(carry):
                t, go, vA, vB = carry
                off = jnp.where(fwd, HALF + t * FB, HALF - (t + 1) * FB)
                pltpu.sync_copy(e_hbm.at[pl.ds(off, FB)], ev0.at[pl.ds(0, FB)])
                pltpu.sync_copy(h_hbm.at[pl.ds(off, FB)], hv0.at[pl.ds(0, FB)])
                pltpu.sync_copy(fc_hbm.at[pl.ds(off, FB)], fcv.at[pl.ds(0, FB)])
                pltpu.sync_copy(sid_hbm.at[pl.ds(off, FB)], sidv0.at[pl.ds(0, FB)])

                def _fgrp(j, fcarry):
                    fvA, fvB, nmatch = fcarry
                    d = pl.ds(j * 16, 16)
                    m = sidv0[d] == sv
                    inv = 1.0 / hv0[d]
                    val = inv * ev0[d] + fcv[d].astype(jnp.float32)
                    zf = jnp.zeros((16,), jnp.float32)
                    fvA = fvA + jnp.where(m, val, zf)
                    fvB = fvB + jnp.where(m, inv, zf)
                    nmatch = nmatch + jnp.sum(m.astype(jnp.int32))
                    return fvA, fvB, nmatch

                vA, vB, nmatch = lax.fori_loop(
                    0, FGRP, _fgrp, (vA, vB, jnp.int32(0)))
                return t + 1, go & (nmatch == FB), vA, vB

            zf16 = jnp.zeros((16,), jnp.float32)
            _, _, vA, vB = lax.while_loop(
                _cond, _body, (jnp.int32(0), jnp.bool_(True), zf16, zf16))

            lane = lax.iota(jnp.int32, 16)
            firstl = (lane == 0).astype(jnp.float32)
            fixidx[:] = sv
            fixA[:] = jnp.sum(vA) * firstl
            fixB[:] = jnp.sum(vB) * firstl
            pltpu.sync_copy(fixA, accA.at[fixidx], add=True)
            pltpu.sync_copy(fixB, accB.at[fixidx], add=True)

    plsc.subcore_barrier()

    sl = pl.ds(s * SLICE, SLICE)
    pltpu.sync_copy(accA.at[sl], ev0.at[pl.ds(0, SLICE)])
    pltpu.sync_copy(accB.at[sl], hv0.at[pl.ds(0, SLICE)])

    @plsc.parallel_loop(0, SLICE // 16, unroll=4)
    def _pm(j):
        d = pl.ds(j * 16, 16)
        ev1[d] = ev0[d] / hv0[d]
    pltpu.sync_copy(ev1.at[pl.ds(0, SLICE)], accA.at[sl])
    plsc.subcore_barrier()

    wb = (first // 8) * 8
    wb_v = jnp.full((16,), wb, jnp.int32)

    @pl.when(local)
    def _pull_pm():
        pltpu.sync_copy(accA.at[pl.ds(wb, W + 8)], pmw)

    in_cps = {0: _start_in(0)}
    st_cps = {}
    for blk in range(NBLK):
        ev, hv, sidv, sin, ssc = bufs[blk % 2]
        for cp in in_cps.pop(blk):
            cp.wait()
        if blk + 1 < NBLK:
            if blk - 1 >= 0:
                for cp in st_cps.pop(blk - 1):
                    cp.wait()
            in_cps[blk + 1] = _start_in(blk + 1)

        @pl.when(local)
        def _local_out():
            @plsc.parallel_loop(0, GRP, unroll=5)
            def _out(j):
                d = pl.ds(j * 16, 16)
                pmg = plsc.load_gather(pmw, [sidv[d] - wb_v])
                inv = 1.0 / hv[d]
                ev[d] = inv * (pmg - ev[d])

        @pl.when(jnp.logical_not(local))
        def _spill_out():
            pltpu.sync_copy(accA.at[sidv], pmbuf)

            @plsc.parallel_loop(0, GRP, unroll=5)
            def _out(j):
                d = pl.ds(j * 16, 16)
                inv = 1.0 / hv[d]
                ev[d] = inv * (pmbuf[d] - ev[d])

        st = base + blk * BLK
        st_cps[blk] = [pltpu.async_copy(ev, out_hbm.at[pl.ds(st, BLK)], ssc)]
    for blk in sorted(st_cps):
        for cp in st_cps.pop(blk):
            cp.wait()


@jax.jit
def kernel(x, formal_charge, segment_ids):
    sid = segment_ids.astype(jnp.int32)
    fc = formal_charge.astype(jnp.int32)
    e = x[:, 0]
    h = x[:, 1]
    charges = _fused(e, h, fc, sid)
    return charges.reshape(-1, 1)


# speedup vs baseline: 1.3594x; 1.2199x over previous
import functools

import jax
import jax.numpy as jnp
from jax import lax
from jax.experimental import pallas as pl
from jax.experimental.pallas import tpu as pltpu
from jax.experimental.pallas import tpu_sc as plsc

N = 1600000
SEG = 50000
NC, NS, L = 2, 16, 16
NW = NC * NS
CHUNK = N // NW
BLK = 10000
NBLK = CHUNK // BLK
GRP = BLK // L
SLICE = 3136
PAD_SEG = NS * SLICE
HALF = N // 2
FB = 2048
FGRP = FB // L

_mesh = plsc.VectorSubcoreMesh(core_axis_name="c", subcore_axis_name="s")
_params = pltpu.CompilerParams(needs_layout_passes=False)


@functools.partial(
    pl.kernel,
    out_type=jax.ShapeDtypeStruct((N,), jnp.float32),
    mesh=_mesh,
    compiler_params=_params,
    scratch_types=[
        pltpu.VMEM((BLK,), jnp.float32),
        pltpu.VMEM((BLK,), jnp.float32),
        pltpu.VMEM((BLK,), jnp.int32),
        pltpu.VMEM((BLK,), jnp.float32),
        pltpu.VMEM((BLK,), jnp.float32),
        pltpu.VMEM((BLK,), jnp.int32),
        pltpu.VMEM((BLK,), jnp.int32),
        pltpu.VMEM((16,), jnp.int32),
        pltpu.VMEM((16,), jnp.float32),
        pltpu.VMEM((16,), jnp.float32),
        pltpu.VMEM((PAD_SEG,), jnp.float32),
        pltpu.VMEM_SHARED((PAD_SEG,), jnp.float32),
        pltpu.VMEM_SHARED((PAD_SEG,), jnp.float32),
        pltpu.VMEM_SHARED((PAD_SEG,), jnp.float32),
        pltpu.SemaphoreType.DMA,
        pltpu.SemaphoreType.DMA,
        pltpu.SemaphoreType.DMA,
        pltpu.SemaphoreType.DMA,
    ],
)
def _fused(e_hbm, h_hbm, fc_hbm, sid_hbm, out_hbm, ev0, hv0, sidv0,
           ev1, hv1, sidv1, fcv, fixidx, fixA, fixB, pmfull,
           accA, accB, pm_sh, sin0, sin1, ssc0, ssc1):
    c = lax.axis_index("c")
    s = lax.axis_index("s")
    wid = c * NS + s
    bufs = [(ev0, hv0, sidv0, sin0, ssc0),
            (ev1, hv1, sidv1, sin1, ssc1)]

    @plsc.parallel_loop(0, SLICE // 16, unroll=4)
    def _zfill(j):
        ev0[pl.ds(j * 16, 16)] = jnp.zeros((16,), jnp.float32)
    pltpu.sync_copy(ev0.at[pl.ds(0, SLICE)], accA.at[pl.ds(s * SLICE, SLICE)])
    pltpu.sync_copy(ev0.at[pl.ds(0, SLICE)], accB.at[pl.ds(s * SLICE, SLICE)])
    plsc.subcore_barrier()

    def _start_in(blk):
        ev, hv, sidv, sin, _ = bufs[blk % 2]
        st = wid * CHUNK + blk * BLK
        return [pltpu.async_copy(e_hbm.at[pl.ds(st, BLK)], ev, sin),
                pltpu.async_copy(h_hbm.at[pl.ds(st, BLK)], hv, sin),
                pltpu.async_copy(sid_hbm.at[pl.ds(st, BLK)], sidv, sin)]

    def _start_fc(blk):
        st = wid * CHUNK + blk * BLK
        return pltpu.async_copy(fc_hbm.at[pl.ds(st, BLK)], fcv, sin0)

    in_cps = {0: _start_in(0)}
    fc_cp = _start_fc(0)
    sc_cps = {}
    for blk in range(NBLK):
        ev, hv, sidv, sin, ssc = bufs[blk % 2]
        for cp in in_cps.pop(blk):
            cp.wait()
        fc_cp.wait()
        if blk + 1 < NBLK:
            if blk - 1 >= 0:
                for cp in sc_cps.pop(blk - 1):
                    cp.wait()
            in_cps[blk + 1] = _start_in(blk + 1)

        @plsc.parallel_loop(0, GRP, unroll=5)
        def _grp(j):
            d = pl.ds(j * 16, 16)
            inv = 1.0 / hv[d]
            ev[d] = inv * ev[d] + fcv[d].astype(jnp.float32)
            hv[d] = inv

        if blk + 1 < NBLK:
            fc_cp = _start_fc(blk + 1)
        sc_cps[blk] = [
            pltpu.async_copy(ev, accA.at[sidv], ssc, add=True),
            pltpu.async_copy(hv, accB.at[sidv], ssc, add=True),
        ]
    for blk in sorted(sc_cps):
        for cp in sc_cps.pop(blk):
            cp.wait()

    plsc.subcore_barrier()

    @pl.when(s == 0)
    def _fixup():
        pltpu.sync_copy(sid_hbm.at[pl.ds(HALF - 8, 16)], fixidx)
        bv = fixidx[pl.ds(0, 16)]
        sid_l = bv[7]
        sid_r = bv[8]

        @pl.when(sid_l == sid_r)
        def _straddle():
            sv = jnp.full((16,), sid_l, jnp.int32)
            fwd = c == 0

            def _cond(carry):
                t, go, _, _ = carry
                return go & (t < HALF // FB)

            def _body(carry):
                t, go, vA, vB = carry
                off = jnp.where(fwd, HALF + t * FB, HALF - (t + 1) * FB)
                pltpu.sync_copy(e_hbm.at[pl.ds(off, FB)], ev0.at[pl.ds(0, FB)])
                pltpu.sync_copy(h_hbm.at[pl.ds(off, FB)], hv0.at[pl.ds(0, FB)])
                pltpu.sync_copy(fc_hbm.at[pl.ds(off, FB)], fcv.at[pl.ds(0, FB)])
                pltpu.sync_copy(sid_hbm.at[pl.ds(off, FB)], sidv0.at[pl.ds(0, FB)])

                def _fgrp(j, fcarry):
                    fvA, fvB, nmatch = fcarry
                    d = pl.ds(j * 16, 16)
                    m = sidv0[d] == sv
                    inv = 1.0 / hv0[d]
                    val = inv * ev0[d] + fcv[d].astype(jnp.float32)
                    zf = jnp.zeros((16,), jnp.float32)
                    fvA = fvA + jnp.where(m, val, zf)
                    fvB = fvB + jnp.where(m, inv, zf)
                    nmatch = nmatch + jnp.sum(m.astype(jnp.int32))
                    return fvA, fvB, nmatch

                vA, vB, nmatch = lax.fori_loop(
                    0, FGRP, _fgrp, (vA, vB, jnp.int32(0)))
                return t + 1, go & (nmatch == FB), vA, vB

            zf16 = jnp.zeros((16,), jnp.float32)
            _, _, vA, vB = lax.while_loop(
                _cond, _body, (jnp.int32(0), jnp.bool_(True), zf16, zf16))

            lane = lax.iota(jnp.int32, 16)
            first = (lane == 0).astype(jnp.float32)
            fixidx[:] = sv
            fixA[:] = jnp.sum(vA) * first
            fixB[:] = jnp.sum(vB) * first
            pltpu.sync_copy(fixA, accA.at[fixidx], add=True)
            pltpu.sync_copy(fixB, accB.at[fixidx], add=True)

    plsc.subcore_barrier()

    sl = pl.ds(s * SLICE, SLICE)
    pltpu.sync_copy(accA.at[sl], ev0.at[pl.ds(0, SLICE)])
    pltpu.sync_copy(accB.at[sl], hv0.at[pl.ds(0, SLICE)])

    @plsc.parallel_loop(0, SLICE // 16, unroll=4)
    def _pm(j):
        d = pl.ds(j * 16, 16)
        ev1[d] = ev0[d] / hv0[d]
    pltpu.sync_copy(ev1.at[pl.ds(0, SLICE)], pm_sh.at[sl])
    plsc.subcore_barrier()

    pltpu.sync_copy(pm_sh, pmfull)

    in_cps = {0: _start_in(0)}
    st_cps = {}
    for blk in range(NBLK):
        ev, hv, sidv, sin, ssc = bufs[blk % 2]
        for cp in in_cps.pop(blk):
            cp.wait()
        if blk + 1 < NBLK:
            if blk - 1 >= 0:
                for cp in st_cps.pop(blk - 1):
                    cp.wait()
            in_cps[blk + 1] = _start_in(blk + 1)

        @plsc.parallel_loop(0, GRP, unroll=5)
        def _out(j):
            d = pl.ds(j * 16, 16)
            pmg = plsc.load_gather(pmfull, [sidv[d]])
            inv = 1.0 / hv[d]
            ev[d] = inv * (pmg - ev[d])

        st = wid * CHUNK + blk * BLK
        st_cps[blk] = [pltpu.async_copy(ev, out_hbm.at[pl.ds(st, BLK)], ssc)]
    for blk in sorted(st_cps):
        for cp in st_cps.pop(blk):
            cp.wait()


@jax.jit
def kernel(x, formal_charge, segment_ids):
    sid = segment_ids.astype(jnp.int32)
    fc = formal_charge.astype(jnp.int32)
    e = x[:, 0]
    h = x[:, 1]
    charges = _fused(e, h, fc, sid)
    return charges.reshape(-1, 1)
